# baseline (speedup 1.0000x reference)
import functools

import jax
import jax.numpy as jnp
from jax import lax
from jax.experimental import pallas as pl
from jax.experimental.pallas import tpu as pltpu
from jax.experimental.pallas import tpu_sc as plsc

_B = 4
_S = 8192
_F = 1024
_NC = 2
_NS = 16
_NW = _NC * _NS
_R = 8
_NSET = 3
_S_SC = 3072
_BS_TC = 512


def _make_sc_add(s_sc):
    rpw = s_sc // _NW
    nblk = rpw // _R

    def body(in_hbm, pe_hbm, out_hbm, *scratch):
        pe_v = list(scratch[0:_NSET])
        in_v = list(scratch[_NSET:2 * _NSET])
        sin = list(scratch[2 * _NSET:3 * _NSET])
        sout = list(scratch[3 * _NSET:4 * _NSET])

        wid = lax.axis_index("s") * _NC + lax.axis_index("c")
        base = wid * rpw

        def issue_in(i, p):
            r0 = base + i * _R
            hs = [pltpu.async_copy(pe_hbm.at[pl.ds(r0, _R)], pe_v[p], sin[p])]
            for b in range(_B):
                hs.append(pltpu.async_copy(
                    in_hbm.at[pl.ds(b * _S + r0, _R)],
                    in_v[p].at[pl.ds(b * _R, _R)],
                    sin[p],
                ))
            return hs

        def issue_out(i, p):
            r0 = base + i * _R
            return [pltpu.async_copy(
                in_v[p].at[pl.ds(b * _R, _R)],
                out_hbm.at[pl.ds(b * s_sc + r0, _R)],
                sout[p],
            ) for b in range(_B)]

        hin = [None] * _NSET
        hout = [None] * _NSET
        hin[0] = issue_in(0, 0)
        hin[1] = issue_in(1, 1)

        for i in range(nblk):
            p = i % _NSET
            for h in hin[p]:
                h.wait()
            nxt = i + 2
            if nxt < nblk:
                p2 = nxt % _NSET
                if hout[p2] is not None:
                    for h in hout[p2]:
                        h.wait()
                    hout[p2] = None
                hin[p2] = issue_in(nxt, p2)

            def chunk(j, carry, p=p):
                o = j * 16
                for r in range(_R):
                    pv = pe_v[p][r, pl.ds(o, 16)]
                    for b in range(_B):
                        in_v[p][b * _R + r, pl.ds(o, 16)] = (
                            in_v[p][b * _R + r, pl.ds(o, 16)] + pv
                        )
                return carry

            lax.fori_loop(0, _F // 16, chunk, 0)
            hout[p] = issue_out(i, p)

        for hs in hout:
            if hs is not None:
                for h in hs:
                    h.wait()

    return pl.kernel(
        body,
        out_type=jax.ShapeDtypeStruct((_B * s_sc, _F), jnp.float32),
        mesh=plsc.VectorSubcoreMesh(core_axis_name="c", subcore_axis_name="s"),
        compiler_params=pltpu.CompilerParams(use_tc_tiling_on_sc=True),
        scratch_types=(
            [pltpu.VMEM((_R, _F), jnp.float32) for _ in range(_NSET)]
            + [pltpu.VMEM((_B * _R, _F), jnp.float32) for _ in range(_NSET)]
            + [pltpu.SemaphoreType.DMA for _ in range(2 * _NSET)]
        ),
    )


_sc_add = _make_sc_add(_S_SC)


def _tc_body(x_ref, pe_ref, o_ref):
    o_ref[...] = x_ref[...] + pe_ref[...][None, :, :]


def _tc_add(inputs, positional_encoding):
    off = _S_SC // _BS_TC
    grid = ((_S - _S_SC) // _BS_TC,)
    return pl.pallas_call(
        _tc_body,
        grid=grid,
        in_specs=[
            pl.BlockSpec((_B, _BS_TC, _F), lambda i: (0, i + off, 0)),
            pl.BlockSpec((_BS_TC, _F), lambda i: (i + off, 0)),
        ],
        out_specs=pl.BlockSpec((_B, _BS_TC, _F), lambda i: (0, i, 0)),
        out_shape=jax.ShapeDtypeStruct((_B, _S - _S_SC, _F), jnp.float32),
    )(inputs, positional_encoding)


def kernel(inputs, positional_encoding):
    b, s, f = inputs.shape
    sc_out = _sc_add(inputs.reshape(b * s, f), positional_encoding)
    tc_out = _tc_add(inputs, positional_encoding)
    return jnp.concatenate(
        [sc_out.reshape(b, _S_SC, f), tc_out], axis=1)
